# baseline (speedup 1.0000x reference)
import functools

import jax
import jax.numpy as jnp
from jax import lax
from jax.experimental import pallas as pl
from jax.experimental.pallas import tpu as pltpu
from jax.experimental.pallas import tpu_sc as plsc

D = 128
N_VOX = 100000
N_PIX = 2 * 4 * 224 * 224
NW = 32
PER_TILE = N_PIX // NW
CHUNK = 128
CHUNKS = PER_TILE // CHUNK
TAB_PAD = 100008
HIST_PAD = 100352
HIST_SLICE = HIST_PAD // 16


def _sc_gather_hist(table, idx3d):
    mesh = plsc.VectorSubcoreMesh(core_axis_name="c", subcore_axis_name="s")

    @functools.partial(
        pl.kernel,
        mesh=mesh,
        out_type=[
            jax.ShapeDtypeStruct((N_PIX, D), jnp.float32),
            jax.ShapeDtypeStruct((2 * HIST_PAD,), jnp.int32),
        ],
        scratch_types=[
            pltpu.VMEM((CHUNKS, CHUNK), jnp.int32),
            pltpu.VMEM((CHUNK, D), jnp.float32),
            pltpu.VMEM((CHUNK, D), jnp.float32),
            pltpu.VMEM((HIST_SLICE,), jnp.int32),
            pltpu.VMEM((CHUNK,), jnp.int32),
            pltpu.VMEM_SHARED((HIST_PAD,), jnp.int32),
            pltpu.SemaphoreType.DMA,
            pltpu.SemaphoreType.DMA,
        ],
    )
    def body(table_hbm, idx_hbm, out_hbm, hist_hbm,
             idx_v, rows0, rows1, zeros_v, ones_v, hist_sh, sem0, sem1):
        c = lax.axis_index("c")
        s = lax.axis_index("s")
        wid = s * 2 + c
        row_base = wid * PER_TILE

        pltpu.sync_copy(idx_hbm.at[wid], idx_v)

        def init_zeros(i, carry):
            zeros_v[pl.ds(i * 16, 16)] = jnp.zeros((16,), jnp.int32)
            return carry

        lax.fori_loop(0, HIST_SLICE // 16, init_zeros, 0)

        def init_ones(i, carry):
            ones_v[pl.ds(i * 16, 16)] = jnp.ones((16,), jnp.int32)
            return carry

        lax.fori_loop(0, CHUNK // 16, init_ones, 0)

        pltpu.sync_copy(zeros_v, hist_sh.at[pl.ds(s * HIST_SLICE, HIST_SLICE)])
        plsc.subcore_barrier()

        pltpu.make_async_copy(table_hbm.at[idx_v.at[0]], rows0, sem0).start()

        def step(g, carry):
            j0 = 2 * g
            j1 = j0 + 1
            pltpu.make_async_copy(table_hbm.at[idx_v.at[j1]], rows1, sem1).start()

            pltpu.make_async_copy(table_hbm.at[idx_v.at[j0]], rows0, sem0).wait()
            pltpu.sync_copy(rows0, out_hbm.at[pl.ds(row_base + j0 * CHUNK, CHUNK)])
            pltpu.sync_copy(ones_v, hist_sh.at[idx_v.at[j0]], add=True)

            @pl.when(g + 1 < CHUNKS // 2)
            def _():
                pltpu.make_async_copy(
                    table_hbm.at[idx_v.at[j0 + 2]], rows0, sem0).start()

            pltpu.make_async_copy(table_hbm.at[idx_v.at[j1]], rows1, sem1).wait()
            pltpu.sync_copy(rows1, out_hbm.at[pl.ds(row_base + j1 * CHUNK, CHUNK)])
            pltpu.sync_copy(ones_v, hist_sh.at[idx_v.at[j1]], add=True)
            return carry

        lax.fori_loop(0, CHUNKS // 2, step, 0)

        plsc.subcore_barrier()
        pltpu.sync_copy(
            hist_sh.at[pl.ds(s * HIST_SLICE, HIST_SLICE)],
            hist_hbm.at[pl.ds(c * HIST_PAD + s * HIST_SLICE, HIST_SLICE)])

    return body(table, idx3d)


def _combine_hist(hist2):
    h3 = hist2.reshape(2, HIST_PAD // D, D)

    def body(h_ref, o_ref):
        o_ref[...] = h_ref[0] + h_ref[1]

    out = pl.pallas_call(
        body,
        out_shape=jax.ShapeDtypeStruct((HIST_PAD // D, D), jnp.int32),
    )(h3)
    return out.reshape(HIST_PAD)


def kernel(features_3d, indexes_image, ignore_label):
    pad = jnp.zeros((TAB_PAD - N_VOX, D), jnp.float32)
    table = jnp.concatenate([features_3d, pad], axis=0)
    idx3d = indexes_image.reshape(NW, CHUNKS, CHUNK)
    projected, hist2 = _sc_gather_hist(table, idx3d)
    counts = _combine_hist(hist2)[:N_VOX]
    return projected, indexes_image, counts
